# baseline (speedup 1.0000x reference)
import functools

import jax
import jax.numpy as jnp
from jax import lax
from jax.experimental import pallas as pl
from jax.experimental.pallas import tpu as pltpu, tpu_sc as plsc

_N_VERTS = 16384
_PIXEL_SCALE = 0.2619
_NC = 2
_NS = 16
_NW = _NC * _NS
_LANES = 16
_C = 2048
_UNROLL = 8

_GATHER_DNUMS = lax.GatherDimensionNumbers(
    offset_dims=(), collapsed_slice_dims=(0,), start_index_map=(0,))


def _take16(v, idx):
    return lax.gather(v, idx[:, None], _GATHER_DNUMS, slice_sizes=(1,),
                      mode=lax.GatherScatterMode.PROMISE_IN_BOUNDS)


def _sc_spmv3_body(n_src, n_val, n_chunks, t_start, *refs):
    it = iter(refs)
    srcs_hbm = [next(it) for _ in range(n_src)]
    rows_hbm = next(it)
    cols_hbm = next(it)
    vals_hbm = [next(it) for _ in range(n_val)]
    rows_t = next(it)
    cols_t = next(it)
    out_hbm = next(it)
    srcs = [next(it) for _ in range(n_src)]
    accs = [next(it) for _ in range(3)]
    rb = [next(it), next(it)]
    cb = [next(it), next(it)]
    vb = [[next(it), next(it)] for _ in range(n_val)]
    sems = [next(it), next(it)]

    cid = lax.axis_index("c")
    sid = lax.axis_index("s")
    wid = sid * _NC + cid
    per_w = n_chunks * _C

    for s_hbm, s in zip(srcs_hbm, srcs):
        pltpu.sync_copy(s_hbm, s)

    zeros = jnp.zeros((_LANES,), jnp.float32)

    @plsc.parallel_loop(0, _N_VERTS, _LANES, unroll=8)
    def _(off):
        off = pl.multiple_of(off, _LANES)
        for a in accs:
            a[pl.ds(off, _LANES)] = zeros

    def issue(ch, b):
        g_ch = wid * n_chunks + ch
        base = wid * per_w + ch * _C
        for j in range(n_val):
            pltpu.async_copy(vals_hbm[j].at[pl.ds(base, _C)], vb[j][b],
                             sems[b])

        @pl.when(g_ch < t_start)
        def _():
            pltpu.async_copy(rows_hbm.at[pl.ds(base, _C)], rb[b], sems[b])
            pltpu.async_copy(cols_hbm.at[pl.ds(base, _C)], cb[b], sems[b])

        @pl.when(g_ch >= t_start)
        def _():
            tbase = (g_ch - t_start) * _C
            pltpu.async_copy(rows_t.at[pl.ds(tbase, _C)], rb[b], sems[b])
            pltpu.async_copy(cols_t.at[pl.ds(tbase, _C)], cb[b], sems[b])

    def drain(b):
        pltpu.make_async_copy(rows_hbm.at[pl.ds(0, _C)], rb[b],
                              sems[b]).wait()
        pltpu.make_async_copy(cols_hbm.at[pl.ds(0, _C)], cb[b],
                              sems[b]).wait()
        for j in range(n_val):
            pltpu.make_async_copy(rows_hbm.at[pl.ds(0, _C)], vb[j][b],
                                  sems[b]).wait()

    iota = lax.iota(jnp.int32, _LANES)
    inx = jnp.minimum(iota + 1, _LANES - 1)
    last_lane = iota == _LANES - 1

    def compute(b):
        @plsc.parallel_loop(0, _C, _LANES, unroll=_UNROLL)
        def _(off):
            off = pl.multiple_of(off, _LANES)
            r = rb[b][pl.ds(off, _LANES)]
            c = cb[b][pl.ds(off, _LANES)]
            r_next = _take16(r, inx)
            is_last = last_lane | (r != r_next)
            is_mid = is_last & (~last_lane)
            if n_src == 1:
                xg0 = plsc.load_gather(srcs[0], [c])
            for j in range(3):
                xg = xg0 if n_src == 1 else plsc.load_gather(srcs[j], [c])
                v = vb[j if n_val == 3 else 0][b][pl.ds(off, _LANES)]
                csum = plsc.cumsum(v * xg)
                plsc.addupdate_scatter(accs[j], [r], csum, mask=is_last)
                plsc.addupdate_scatter(accs[j], [r_next], -csum, mask=is_mid)

    issue(0, 0)
    issue(1, 1)

    def pair_body(g, carry):
        ch0 = g * 2
        drain(0)
        compute(0)

        @pl.when(ch0 + 2 < n_chunks)
        def _():
            issue(ch0 + 2, 0)

        ch1 = ch0 + 1

        @pl.when(ch1 < n_chunks)
        def _():
            drain(1)
            compute(1)

        @pl.when(ch1 + 2 < n_chunks)
        def _():
            issue(ch1 + 2, 1)

        return carry

    lax.fori_loop(0, (n_chunks + 1) // 2, pair_body, 0)

    for j in range(3):
        pltpu.sync_copy(accs[j],
                        out_hbm.at[pl.ds((wid * 3 + j) * _N_VERTS, _N_VERTS)])


def _sc_phase(n_chunks, t_start, srcs, rows, cols, vals, rows_t, cols_t):
    n_src, n_val = len(srcs), len(vals)
    mesh = plsc.VectorSubcoreMesh(core_axis_name="c", subcore_axis_name="s")
    body = functools.partial(_sc_spmv3_body, n_src, n_val, n_chunks, t_start)
    return pl.kernel(
        body,
        out_type=jax.ShapeDtypeStruct((_NW * 3 * _N_VERTS,), jnp.float32),
        mesh=mesh,
        compiler_params=pltpu.CompilerParams(needs_layout_passes=False),
        scratch_types=(
            [pltpu.VMEM((_N_VERTS,), jnp.float32) for _ in range(n_src)]
            + [pltpu.VMEM((_N_VERTS,), jnp.float32) for _ in range(3)]
            + [pltpu.VMEM((_C,), jnp.int32) for _ in range(4)]
            + [pltpu.VMEM((_C,), jnp.float32) for _ in range(2 * n_val)]
            + [pltpu.SemaphoreType.DMA, pltpu.SemaphoreType.DMA]
        ),
    )(*srcs, rows, cols, *vals, rows_t, cols_t)


def _tc_split_vals(op_vals, nnzp):
    nnz = op_vals.shape[1]
    blk = 65536
    grid = nnzp // blk

    def body(v_ref, o0, o1, o2, o3):
        g = pl.program_id(0)
        pos = g * blk + lax.broadcasted_iota(jnp.int32, (blk,), 0)
        m = pos < nnz
        for j, o in enumerate((o0, o1, o2, o3)):
            o[...] = jnp.where(m, v_ref[j, :], 0.0)

    out = jax.ShapeDtypeStruct((nnzp,), jnp.float32)
    return pl.pallas_call(
        body,
        grid=(grid,),
        in_specs=[pl.BlockSpec((4, blk), lambda g: (0, g))],
        out_specs=[pl.BlockSpec((blk,), lambda g: (g,))] * 4,
        out_shape=[out] * 4,
    )(op_vals)


def _tc_reduce(partials, scale, transpose=False):
    def body(p_ref, o_ref):
        s = jnp.sum(p_ref[...], axis=0) * scale
        o_ref[...] = s.T if transpose else s

    blk = _N_VERTS // 8
    if transpose:
        out_spec = pl.BlockSpec((blk, 3), lambda g: (g, 0))
        out_shape = jax.ShapeDtypeStruct((_N_VERTS, 3), jnp.float32)
    else:
        out_spec = pl.BlockSpec((3, blk), lambda g: (0, g))
        out_shape = jax.ShapeDtypeStruct((3, _N_VERTS), jnp.float32)
    return pl.pallas_call(
        body,
        grid=(8,),
        in_specs=[pl.BlockSpec((_NW, 3, blk), lambda g: (0, 0, g))],
        out_specs=out_spec,
        out_shape=out_shape,
    )(partials)


def kernel(X, op_rows, op_cols, op_vals):
    nnz = op_rows.shape[0]
    n_chunks = -(-nnz // (_NW * _C))
    t_start = nnz // _C
    n_tail = _NW * n_chunks - t_start
    tpad = t_start * _C + n_tail * _C - nnz

    def tail(a):
        return jnp.pad(a[t_start * _C:], (0, tpad))

    x_flat = X.reshape(-1)
    nnzp = _NW * n_chunks * _C
    v0, v1, v2, v3 = _tc_split_vals(op_vals, nnzp)
    rows_t, cols_t = tail(op_rows), tail(op_cols)
    partials_a = _sc_phase(n_chunks, t_start, [x_flat], op_rows, op_cols,
                           [v1, v2, v3], rows_t, cols_t)
    L = _tc_reduce(partials_a.reshape(_NW, 3, _N_VERTS), 1.0)
    partials_b = _sc_phase(n_chunks, t_start, [L[0], L[1], L[2]],
                           op_rows, op_cols, [v0], rows_t, cols_t)
    return _tc_reduce(partials_b.reshape(_NW, 3, _N_VERTS),
                      1.0 / _PIXEL_SCALE, transpose=True)


# speedup vs baseline: 1.0130x; 1.0130x over previous
import functools

import jax
import jax.numpy as jnp
from jax import lax
from jax.experimental import pallas as pl
from jax.experimental.pallas import tpu as pltpu, tpu_sc as plsc

_N_VERTS = 16384
_PIXEL_SCALE = 0.2619
_NC = 2
_NS = 16
_NW = _NC * _NS
_LANES = 16
_C = 2048
_UNROLL = 4

_GATHER_DNUMS = lax.GatherDimensionNumbers(
    offset_dims=(), collapsed_slice_dims=(0,), start_index_map=(0,))


def _take16(v, idx):
    return lax.gather(v, idx[:, None], _GATHER_DNUMS, slice_sizes=(1,),
                      mode=lax.GatherScatterMode.PROMISE_IN_BOUNDS)


def _sc_spmv3_body(n_src, n_val, n_chunks, t_start, *refs):
    it = iter(refs)
    srcs_hbm = [next(it) for _ in range(n_src)]
    rows_hbm = next(it)
    cols_hbm = next(it)
    vals_hbm = [next(it) for _ in range(n_val)]
    rows_t = next(it)
    cols_t = next(it)
    out_hbm = next(it)
    srcs = [next(it) for _ in range(n_src)]
    accs = [next(it) for _ in range(3)]
    rb = [next(it), next(it)]
    cb = [next(it), next(it)]
    vb = [[next(it), next(it)] for _ in range(n_val)]
    sems = [next(it), next(it)]

    cid = lax.axis_index("c")
    sid = lax.axis_index("s")
    wid = sid * _NC + cid
    per_w = n_chunks * _C

    for s_hbm, s in zip(srcs_hbm, srcs):
        pltpu.sync_copy(s_hbm, s)

    zeros = jnp.zeros((_LANES,), jnp.float32)

    @plsc.parallel_loop(0, _N_VERTS, _LANES, unroll=8)
    def _(off):
        off = pl.multiple_of(off, _LANES)
        for a in accs:
            a[pl.ds(off, _LANES)] = zeros

    def issue(ch, b):
        g_ch = wid * n_chunks + ch
        base = wid * per_w + ch * _C
        for j in range(n_val):
            pltpu.async_copy(vals_hbm[j].at[pl.ds(base, _C)], vb[j][b],
                             sems[b])

        @pl.when(g_ch < t_start)
        def _():
            pltpu.async_copy(rows_hbm.at[pl.ds(base, _C)], rb[b], sems[b])
            pltpu.async_copy(cols_hbm.at[pl.ds(base, _C)], cb[b], sems[b])

        @pl.when(g_ch >= t_start)
        def _():
            tbase = (g_ch - t_start) * _C
            pltpu.async_copy(rows_t.at[pl.ds(tbase, _C)], rb[b], sems[b])
            pltpu.async_copy(cols_t.at[pl.ds(tbase, _C)], cb[b], sems[b])

    def drain(b):
        pltpu.make_async_copy(rows_hbm.at[pl.ds(0, _C)], rb[b],
                              sems[b]).wait()
        pltpu.make_async_copy(cols_hbm.at[pl.ds(0, _C)], cb[b],
                              sems[b]).wait()
        for j in range(n_val):
            pltpu.make_async_copy(rows_hbm.at[pl.ds(0, _C)], vb[j][b],
                                  sems[b]).wait()

    iota = lax.iota(jnp.int32, _LANES)
    inx = jnp.minimum(iota + 1, _LANES - 1)
    last_lane = iota == _LANES - 1

    def compute(b):
        @plsc.parallel_loop(0, _C, _LANES, unroll=_UNROLL)
        def _(off):
            off = pl.multiple_of(off, _LANES)
            r = rb[b][pl.ds(off, _LANES)]
            c = cb[b][pl.ds(off, _LANES)]
            r_next = _take16(r, inx)
            is_last = last_lane | (r != r_next)
            is_mid = is_last & (~last_lane)
            if n_src == 1:
                xg0 = plsc.load_gather(srcs[0], [c])
            for j in range(3):
                xg = xg0 if n_src == 1 else plsc.load_gather(srcs[j], [c])
                v = vb[j if n_val == 3 else 0][b][pl.ds(off, _LANES)]
                csum = plsc.cumsum(v * xg)
                plsc.addupdate_scatter(accs[j], [r], csum, mask=is_last)
                plsc.addupdate_scatter(accs[j], [r_next], -csum, mask=is_mid)

    issue(0, 0)
    issue(1, 1)

    def pair_body(g, carry):
        ch0 = g * 2
        drain(0)
        compute(0)

        @pl.when(ch0 + 2 < n_chunks)
        def _():
            issue(ch0 + 2, 0)

        ch1 = ch0 + 1

        @pl.when(ch1 < n_chunks)
        def _():
            drain(1)
            compute(1)

        @pl.when(ch1 + 2 < n_chunks)
        def _():
            issue(ch1 + 2, 1)

        return carry

    lax.fori_loop(0, (n_chunks + 1) // 2, pair_body, 0)

    for j in range(3):
        pltpu.sync_copy(accs[j],
                        out_hbm.at[pl.ds((wid * 3 + j) * _N_VERTS, _N_VERTS)])


def _sc_phase(n_chunks, t_start, srcs, rows, cols, vals, rows_t, cols_t):
    n_src, n_val = len(srcs), len(vals)
    mesh = plsc.VectorSubcoreMesh(core_axis_name="c", subcore_axis_name="s")
    body = functools.partial(_sc_spmv3_body, n_src, n_val, n_chunks, t_start)
    return pl.kernel(
        body,
        out_type=jax.ShapeDtypeStruct((_NW * 3 * _N_VERTS,), jnp.float32),
        mesh=mesh,
        compiler_params=pltpu.CompilerParams(needs_layout_passes=False),
        scratch_types=(
            [pltpu.VMEM((_N_VERTS,), jnp.float32) for _ in range(n_src)]
            + [pltpu.VMEM((_N_VERTS,), jnp.float32) for _ in range(3)]
            + [pltpu.VMEM((_C,), jnp.int32) for _ in range(4)]
            + [pltpu.VMEM((_C,), jnp.float32) for _ in range(2 * n_val)]
            + [pltpu.SemaphoreType.DMA, pltpu.SemaphoreType.DMA]
        ),
    )(*srcs, rows, cols, *vals, rows_t, cols_t)


def _tc_split_vals(op_vals, nnzp):
    nnz = op_vals.shape[1]
    blk = 65536
    grid = nnzp // blk

    def body(v_ref, o0, o1, o2, o3):
        g = pl.program_id(0)
        pos = g * blk + lax.broadcasted_iota(jnp.int32, (blk,), 0)
        m = pos < nnz
        for j, o in enumerate((o0, o1, o2, o3)):
            o[...] = jnp.where(m, v_ref[j, :], 0.0)

    out = jax.ShapeDtypeStruct((nnzp,), jnp.float32)
    return pl.pallas_call(
        body,
        grid=(grid,),
        in_specs=[pl.BlockSpec((4, blk), lambda g: (0, g))],
        out_specs=[pl.BlockSpec((blk,), lambda g: (g,))] * 4,
        out_shape=[out] * 4,
    )(op_vals)


def _tc_reduce(partials, scale, transpose=False):
    def body(p_ref, o_ref):
        s = jnp.sum(p_ref[...], axis=0) * scale
        o_ref[...] = s.T if transpose else s

    blk = _N_VERTS // 8
    if transpose:
        out_spec = pl.BlockSpec((blk, 3), lambda g: (g, 0))
        out_shape = jax.ShapeDtypeStruct((_N_VERTS, 3), jnp.float32)
    else:
        out_spec = pl.BlockSpec((3, blk), lambda g: (0, g))
        out_shape = jax.ShapeDtypeStruct((3, _N_VERTS), jnp.float32)
    return pl.pallas_call(
        body,
        grid=(8,),
        in_specs=[pl.BlockSpec((_NW, 3, blk), lambda g: (0, 0, g))],
        out_specs=out_spec,
        out_shape=out_shape,
    )(partials)


def kernel(X, op_rows, op_cols, op_vals):
    nnz = op_rows.shape[0]
    n_chunks = -(-nnz // (_NW * _C))
    t_start = nnz // _C
    n_tail = _NW * n_chunks - t_start
    tpad = t_start * _C + n_tail * _C - nnz

    def tail(a):
        return jnp.pad(a[t_start * _C:], (0, tpad))

    x_flat = X.reshape(-1)
    nnzp = _NW * n_chunks * _C
    v0, v1, v2, v3 = _tc_split_vals(op_vals, nnzp)
    rows_t, cols_t = tail(op_rows), tail(op_cols)
    partials_a = _sc_phase(n_chunks, t_start, [x_flat], op_rows, op_cols,
                           [v1, v2, v3], rows_t, cols_t)
    L = _tc_reduce(partials_a.reshape(_NW, 3, _N_VERTS), 1.0)
    partials_b = _sc_phase(n_chunks, t_start, [L[0], L[1], L[2]],
                           op_rows, op_cols, [v0], rows_t, cols_t)
    return _tc_reduce(partials_b.reshape(_NW, 3, _N_VERTS),
                      1.0 / _PIXEL_SCALE, transpose=True)


# speedup vs baseline: 1.2327x; 1.2169x over previous
import functools

import jax
import jax.numpy as jnp
from jax import lax
from jax.experimental import pallas as pl
from jax.experimental.pallas import tpu as pltpu, tpu_sc as plsc

_N_VERTS = 16384
_PIXEL_SCALE = 0.2619
_NC = 2
_NS = 16
_NW = _NC * _NS
_LANES = 16
_C = 2048
_UNROLL = 4

_GATHER_DNUMS = lax.GatherDimensionNumbers(
    offset_dims=(), collapsed_slice_dims=(0,), start_index_map=(0,))


def _take16(v, idx):
    return lax.gather(v, idx[:, None], _GATHER_DNUMS, slice_sizes=(1,),
                      mode=lax.GatherScatterMode.PROMISE_IN_BOUNDS)


def _sc_spmv3_body(n_src, n_val, n_chunks, t_start, *refs):
    it = iter(refs)
    srcs_hbm = [next(it) for _ in range(n_src)]
    rows_hbm = next(it)
    cols_hbm = next(it)
    vals_hbm = [next(it) for _ in range(n_val)]
    rows_t = next(it)
    cols_t = next(it)
    out_hbm = next(it)
    srcs = [next(it) for _ in range(n_src)]
    accs = [next(it) for _ in range(3)]
    rb = [next(it), next(it)]
    cb = [next(it), next(it)]
    vb = [[next(it), next(it)] for _ in range(n_val)]
    sems = [next(it), next(it)]

    cid = lax.axis_index("c")
    sid = lax.axis_index("s")
    wid = sid * _NC + cid
    per_w = n_chunks * _C

    for s_hbm, s in zip(srcs_hbm, srcs):
        pltpu.sync_copy(s_hbm, s)

    zeros = jnp.zeros((_LANES,), jnp.float32)

    @plsc.parallel_loop(0, _N_VERTS, _LANES, unroll=8)
    def _(off):
        off = pl.multiple_of(off, _LANES)
        for a in accs:
            a[pl.ds(off, _LANES)] = zeros

    def issue(ch, b):
        g_ch = wid * n_chunks + ch
        base = wid * per_w + ch * _C
        for j in range(n_val):
            pltpu.async_copy(vals_hbm[j].at[pl.ds(base, _C)], vb[j][b],
                             sems[b])

        @pl.when(g_ch < t_start)
        def _():
            pltpu.async_copy(rows_hbm.at[pl.ds(base, _C)], rb[b], sems[b])
            pltpu.async_copy(cols_hbm.at[pl.ds(base, _C)], cb[b], sems[b])

        @pl.when(g_ch >= t_start)
        def _():
            tbase = (g_ch - t_start) * _C
            pltpu.async_copy(rows_t.at[pl.ds(tbase, _C)], rb[b], sems[b])
            pltpu.async_copy(cols_t.at[pl.ds(tbase, _C)], cb[b], sems[b])

    def drain(b):
        pltpu.make_async_copy(rows_hbm.at[pl.ds(0, _C)], rb[b],
                              sems[b]).wait()
        pltpu.make_async_copy(cols_hbm.at[pl.ds(0, _C)], cb[b],
                              sems[b]).wait()
        for j in range(n_val):
            pltpu.make_async_copy(rows_hbm.at[pl.ds(0, _C)], vb[j][b],
                                  sems[b]).wait()

    iota = lax.iota(jnp.int32, _LANES)
    inx = jnp.minimum(iota + 1, _LANES - 1)
    last_lane = iota == _LANES - 1

    def compute(b):
        @plsc.parallel_loop(0, _C, _LANES, unroll=_UNROLL)
        def _(off):
            off = pl.multiple_of(off, _LANES)
            r = rb[b][pl.ds(off, _LANES)]
            c = cb[b][pl.ds(off, _LANES)]
            r_next = _take16(r, inx)
            is_last = last_lane | (r != r_next)
            is_mid = is_last & (~last_lane)
            if n_src == 1:
                xg0 = plsc.load_gather(srcs[0], [c])
            for j in range(3):
                xg = xg0 if n_src == 1 else plsc.load_gather(srcs[j], [c])
                v = vb[j if n_val == 3 else 0][b][pl.ds(off, _LANES)]
                csum = plsc.cumsum(v * xg)
                plsc.addupdate_scatter(accs[j], [r], csum, mask=is_last)
                plsc.addupdate_scatter(accs[j], [r_next], -csum, mask=is_mid)

    issue(0, 0)
    issue(1, 1)

    def pair_body(g, carry):
        ch0 = g * 2
        drain(0)
        compute(0)

        @pl.when(ch0 + 2 < n_chunks)
        def _():
            issue(ch0 + 2, 0)

        ch1 = ch0 + 1

        @pl.when(ch1 < n_chunks)
        def _():
            drain(1)
            compute(1)

        @pl.when(ch1 + 2 < n_chunks)
        def _():
            issue(ch1 + 2, 1)

        return carry

    lax.fori_loop(0, (n_chunks + 1) // 2, pair_body, 0)

    for j in range(3):
        pltpu.sync_copy(accs[j],
                        out_hbm.at[pl.ds((wid * 3 + j) * _N_VERTS, _N_VERTS)])


def _sc_phase(n_chunks, t_start, srcs, rows, cols, vals, rows_t, cols_t):
    n_src, n_val = len(srcs), len(vals)
    mesh = plsc.VectorSubcoreMesh(core_axis_name="c", subcore_axis_name="s")
    body = functools.partial(_sc_spmv3_body, n_src, n_val, n_chunks, t_start)
    return pl.kernel(
        body,
        out_type=jax.ShapeDtypeStruct((_NW * 3 * _N_VERTS,), jnp.float32),
        mesh=mesh,
        compiler_params=pltpu.CompilerParams(needs_layout_passes=False),
        scratch_types=(
            [pltpu.VMEM((_N_VERTS,), jnp.float32) for _ in range(n_src)]
            + [pltpu.VMEM((_N_VERTS,), jnp.float32) for _ in range(3)]
            + [pltpu.VMEM((_C,), jnp.int32) for _ in range(4)]
            + [pltpu.VMEM((_C,), jnp.float32) for _ in range(2 * n_val)]
            + [pltpu.SemaphoreType.DMA, pltpu.SemaphoreType.DMA]
        ),
    )(*srcs, rows, cols, *vals, rows_t, cols_t)


def _tc_split_vals(op_vals, nnzp):
    nnz = op_vals.shape[1]
    blk = nnzp // 8
    grid = 8

    def body(v_ref, o0, o1, o2, o3):
        g = pl.program_id(0)
        pos = g * blk + lax.broadcasted_iota(jnp.int32, (blk,), 0)
        m = pos < nnz
        for j, o in enumerate((o0, o1, o2, o3)):
            o[...] = jnp.where(m, v_ref[j, :], 0.0)

    out = jax.ShapeDtypeStruct((nnzp,), jnp.float32)
    return pl.pallas_call(
        body,
        grid=(grid,),
        in_specs=[pl.BlockSpec((4, blk), lambda g: (0, g))],
        out_specs=[pl.BlockSpec((blk,), lambda g: (g,))] * 4,
        out_shape=[out] * 4,
    )(op_vals)


def _tc_reduce(partials_flat, scale):
    def body(p_ref, o_ref):
        for j in range(3):
            a = jnp.zeros((_N_VERTS,), jnp.float32)
            for w in range(_NW):
                a = a + p_ref[pl.ds((w * 3 + j) * _N_VERTS, _N_VERTS)]
            o_ref[j, :] = a * scale

    return pl.pallas_call(
        body,
        out_shape=jax.ShapeDtypeStruct((3, _N_VERTS), jnp.float32),
    )(partials_flat)


def kernel(X, op_rows, op_cols, op_vals):
    nnz = op_rows.shape[0]
    n_chunks = -(-nnz // (_NW * _C))
    t_start = nnz // _C
    n_tail = _NW * n_chunks - t_start
    tpad = t_start * _C + n_tail * _C - nnz

    def tail(a):
        return jnp.pad(a[t_start * _C:], (0, tpad))

    x_flat = X.reshape(-1)
    nnzp = _NW * n_chunks * _C
    v0, v1, v2, v3 = _tc_split_vals(op_vals, nnzp)
    rows_t, cols_t = tail(op_rows), tail(op_cols)
    partials_a = _sc_phase(n_chunks, t_start, [x_flat], op_rows, op_cols,
                           [v1, v2, v3], rows_t, cols_t)
    L = _tc_reduce(partials_a, 1.0)
    partials_b = _sc_phase(n_chunks, t_start, [L[0], L[1], L[2]],
                           op_rows, op_cols, [v0], rows_t, cols_t)
    grad = _tc_reduce(partials_b, 1.0 / _PIXEL_SCALE)
    return grad.T
